# baseline (device time: 71232 ns/iter reference)
import jax
import jax.numpy as jnp
from jax import lax
from jax.experimental import pallas as pl
from jax.experimental.pallas import tpu as pltpu

N_DEV = 16
M_PER = 256
K = 4096
N_PER = 512
F8_MAX = 448.0
N_BF = 12


def kernel(x, w_mat):
    m_per, k = x.shape
    _, n = w_mat.shape
    assert (m_per, k, n) == (M_PER, K, N_PER * N_DEV)

    def body(x_ref, w_hbm, out_ref, w_vmem, y_bf, y32, yq, recv8, recv_bf,
             amax_buf, w_sems, amax_send_sems, amax_recv_sems, bf_send_sems,
             bf_recv_sems, a2a_send_sems, a2a_recv_sems):
        my = lax.axis_index("i")

        barrier_sem = pltpu.get_barrier_semaphore()
        for d in range(1, N_DEV):
            pl.semaphore_signal(
                barrier_sem, inc=1,
                device_id=(lax.rem(my + d, N_DEV),),
                device_id_type=pl.DeviceIdType.MESH,
            )

        def w_copy(t, slot):
            j = lax.rem(my + t, N_DEV)
            return pltpu.make_async_copy(
                w_hbm.at[:, pl.ds(j * N_PER, N_PER)],
                w_vmem.at[slot],
                w_sems.at[slot],
            )

        w_copy(0, 0).start()
        local_max = jnp.float32(0.0)
        bf_sends = []
        for t in range(N_DEV):
            slot = t % 2
            if t + 1 < N_DEV:
                w_copy(t + 1, 1 - slot).start()
            w_copy(t, slot).wait()
            j = lax.rem(my + t, N_DEV)
            yblk = jnp.maximum(
                jnp.dot(x_ref[:, :], w_vmem[slot],
                        preferred_element_type=jnp.float32),
                0.0,
            )
            local_max = jnp.maximum(local_max, jnp.max(yblk))
            y32[j] = yblk
            if t == 0:
                pl.semaphore_wait(barrier_sem, N_DEV - 1)
            elif t < N_BF:
                y_bf[t] = yblk.astype(jnp.bfloat16)
                rdma = pltpu.make_async_remote_copy(
                    src_ref=y_bf.at[t],
                    dst_ref=recv_bf.at[t],
                    send_sem=bf_send_sems.at[t],
                    recv_sem=bf_recv_sems.at[t],
                    device_id=(j,),
                    device_id_type=pl.DeviceIdType.MESH,
                )
                rdma.start()
                bf_sends.append(rdma)

        amax_buf[0] = jnp.full((8, 128), local_max, dtype=jnp.float32)
        amax_sends = []
        for d in range(1, N_DEV):
            rdma = pltpu.make_async_remote_copy(
                src_ref=amax_buf.at[0],
                dst_ref=amax_buf.at[d],
                send_sem=amax_send_sems.at[d],
                recv_sem=amax_recv_sems.at[d],
                device_id=(lax.rem(my + d, N_DEV),),
                device_id_type=pl.DeviceIdType.MESH,
            )
            rdma.start()
            amax_sends.append(rdma)
        for d in range(1, N_DEV):
            pltpu.make_async_remote_copy(
                src_ref=amax_buf.at[0],
                dst_ref=amax_buf.at[d],
                send_sem=amax_send_sems.at[d],
                recv_sem=amax_recv_sems.at[d],
                device_id=(my,),
                device_id_type=pl.DeviceIdType.MESH,
            ).wait_recv()
        gmax = jnp.max(amax_buf[...])
        inv_scale = F8_MAX / gmax
        scale = gmax / F8_MAX

        def quant_dequant(v):
            q = (v.astype(jnp.float32) * inv_scale).astype(jnp.float8_e4m3fn)
            return (q.astype(jnp.float32) * scale).astype(jnp.bfloat16)

        a2a_sends = []
        for t in range(N_BF, N_DEV):
            dst = lax.rem(my + t, N_DEV)
            yq[dst] = (y32[dst] * inv_scale).astype(jnp.float8_e4m3fn)
            rdma = pltpu.make_async_remote_copy(
                src_ref=yq.at[dst],
                dst_ref=recv8.at[my],
                send_sem=a2a_send_sems.at[t],
                recv_sem=a2a_recv_sems.at[t],
                device_id=(dst,),
                device_id_type=pl.DeviceIdType.MESH,
            )
            rdma.start()
            a2a_sends.append(rdma)

        out_ref[pl.ds(my * M_PER, M_PER), :] = quant_dequant(y32[my])

        for t in range(1, N_BF):
            src = lax.rem(my - t + N_DEV, N_DEV)
            pltpu.make_async_remote_copy(
                src_ref=y_bf.at[t],
                dst_ref=recv_bf.at[t],
                send_sem=bf_send_sems.at[t],
                recv_sem=bf_recv_sems.at[t],
                device_id=(my,),
                device_id_type=pl.DeviceIdType.MESH,
            ).wait_recv()
            out_ref[pl.ds(src * M_PER, M_PER), :] = quant_dequant(recv_bf[t])

        for t in range(N_BF, N_DEV):
            src = lax.rem(my - t + N_DEV, N_DEV)
            pltpu.make_async_remote_copy(
                src_ref=yq.at[src],
                dst_ref=recv8.at[src],
                send_sem=a2a_send_sems.at[t],
                recv_sem=a2a_recv_sems.at[t],
                device_id=(my,),
                device_id_type=pl.DeviceIdType.MESH,
            ).wait_recv()
            out_ref[pl.ds(src * M_PER, M_PER), :] = (
                recv8[src].astype(jnp.float32) * scale
            ).astype(jnp.bfloat16)

        for rdma in bf_sends + amax_sends + a2a_sends:
            rdma.wait_send()

    return pl.pallas_call(
        body,
        out_shape=jax.ShapeDtypeStruct((N_DEV * M_PER, N_PER), jnp.bfloat16),
        in_specs=[
            pl.BlockSpec(memory_space=pltpu.VMEM),
            pl.BlockSpec(memory_space=pltpu.MemorySpace.HBM),
        ],
        out_specs=pl.BlockSpec(memory_space=pltpu.VMEM),
        scratch_shapes=[
            pltpu.VMEM((2, K, N_PER), jnp.float32),
            pltpu.VMEM((N_BF, M_PER, N_PER), jnp.bfloat16),
            pltpu.VMEM((N_DEV, M_PER, N_PER), jnp.float32),
            pltpu.VMEM((N_DEV, M_PER, N_PER), jnp.float8_e4m3fn),
            pltpu.VMEM((N_DEV, M_PER, N_PER), jnp.float8_e4m3fn),
            pltpu.VMEM((N_BF, M_PER, N_PER), jnp.bfloat16),
            pltpu.VMEM((N_DEV, 8, 128), jnp.float32),
            pltpu.SemaphoreType.DMA((2,)),
            pltpu.SemaphoreType.DMA((N_DEV,)),
            pltpu.SemaphoreType.DMA((N_DEV,)),
            pltpu.SemaphoreType.DMA((N_BF,)),
            pltpu.SemaphoreType.DMA((N_BF,)),
            pltpu.SemaphoreType.DMA((N_DEV,)),
            pltpu.SemaphoreType.DMA((N_DEV,)),
        ],
        compiler_params=pltpu.CompilerParams(
            collective_id=0, vmem_limit_bytes=48 * 1024 * 1024,
        ),
    )(x, w_mat)


# device time: 66218 ns/iter; 1.0757x vs baseline; 1.0757x over previous
import jax
import jax.numpy as jnp
from jax import lax
from jax.experimental import pallas as pl
from jax.experimental.pallas import tpu as pltpu

N_DEV = 16
M_PER = 256
K = 4096
N_PER = 512
F8_MAX = 448.0
N_BF = 10


def kernel(x, w_mat):
    m_per, k = x.shape
    _, n = w_mat.shape
    assert (m_per, k, n) == (M_PER, K, N_PER * N_DEV)

    def body(x_ref, w_hbm, out_ref, w_vmem, y_bf, y32, yq, recv8, recv_bf,
             amax_buf, w_sems, amax_send_sems, amax_recv_sems, bf_send_sems,
             bf_recv_sems, a2a_send_sems, a2a_recv_sems):
        my = lax.axis_index("i")

        barrier_sem = pltpu.get_barrier_semaphore()
        for d in range(1, N_DEV):
            pl.semaphore_signal(
                barrier_sem, inc=1,
                device_id=(lax.rem(my + d, N_DEV),),
                device_id_type=pl.DeviceIdType.MESH,
            )

        def w_copy(t, slot):
            j = lax.rem(my + t, N_DEV)
            return pltpu.make_async_copy(
                w_hbm.at[:, pl.ds(j * N_PER, N_PER)],
                w_vmem.at[slot],
                w_sems.at[slot],
            )

        w_copy(0, 0).start()
        local_max = jnp.float32(0.0)
        bf_sends = []
        for t in range(N_DEV):
            slot = t % 2
            if t + 1 < N_DEV:
                w_copy(t + 1, 1 - slot).start()
            w_copy(t, slot).wait()
            j = lax.rem(my + t, N_DEV)
            yblk = jnp.maximum(
                jnp.dot(x_ref[:, :], w_vmem[slot],
                        preferred_element_type=jnp.float32),
                0.0,
            )
            local_max = jnp.maximum(local_max, jnp.max(yblk))
            y32[j] = yblk
            if t == 0:
                pl.semaphore_wait(barrier_sem, N_DEV - 1)
            elif t < N_BF:
                y_bf[t] = yblk.astype(jnp.bfloat16)
                rdma = pltpu.make_async_remote_copy(
                    src_ref=y_bf.at[t],
                    dst_ref=recv_bf.at[t],
                    send_sem=bf_send_sems.at[t],
                    recv_sem=bf_recv_sems.at[t],
                    device_id=(j,),
                    device_id_type=pl.DeviceIdType.MESH,
                )
                rdma.start()
                bf_sends.append(rdma)

        amax_buf[0] = jnp.full((8, 128), local_max, dtype=jnp.float32)
        amax_sends = []
        for d in range(1, N_DEV):
            rdma = pltpu.make_async_remote_copy(
                src_ref=amax_buf.at[0],
                dst_ref=amax_buf.at[d],
                send_sem=amax_send_sems.at[d],
                recv_sem=amax_recv_sems.at[d],
                device_id=(lax.rem(my + d, N_DEV),),
                device_id_type=pl.DeviceIdType.MESH,
            )
            rdma.start()
            amax_sends.append(rdma)
        for d in range(1, N_DEV):
            pltpu.make_async_remote_copy(
                src_ref=amax_buf.at[0],
                dst_ref=amax_buf.at[d],
                send_sem=amax_send_sems.at[d],
                recv_sem=amax_recv_sems.at[d],
                device_id=(my,),
                device_id_type=pl.DeviceIdType.MESH,
            ).wait_recv()
        gmax = jnp.max(amax_buf[...])
        inv_scale = F8_MAX / gmax
        scale = gmax / F8_MAX

        def quant_dequant(v):
            q = (v.astype(jnp.float32) * inv_scale).astype(jnp.float8_e4m3fn)
            return (q.astype(jnp.float32) * scale).astype(jnp.bfloat16)

        a2a_sends = []
        for t in range(N_BF, N_DEV):
            dst = lax.rem(my + t, N_DEV)
            yq[dst] = (y32[dst] * inv_scale).astype(jnp.float8_e4m3fn)
            rdma = pltpu.make_async_remote_copy(
                src_ref=yq.at[dst],
                dst_ref=recv8.at[my],
                send_sem=a2a_send_sems.at[t],
                recv_sem=a2a_recv_sems.at[t],
                device_id=(dst,),
                device_id_type=pl.DeviceIdType.MESH,
            )
            rdma.start()
            a2a_sends.append(rdma)

        out_ref[pl.ds(my * M_PER, M_PER), :] = quant_dequant(y32[my])

        for t in range(1, N_BF):
            src = lax.rem(my - t + N_DEV, N_DEV)
            pltpu.make_async_remote_copy(
                src_ref=y_bf.at[t],
                dst_ref=recv_bf.at[t],
                send_sem=bf_send_sems.at[t],
                recv_sem=bf_recv_sems.at[t],
                device_id=(my,),
                device_id_type=pl.DeviceIdType.MESH,
            ).wait_recv()
            out_ref[pl.ds(src * M_PER, M_PER), :] = quant_dequant(recv_bf[t])

        for t in range(N_BF, N_DEV):
            src = lax.rem(my - t + N_DEV, N_DEV)
            pltpu.make_async_remote_copy(
                src_ref=yq.at[src],
                dst_ref=recv8.at[src],
                send_sem=a2a_send_sems.at[t],
                recv_sem=a2a_recv_sems.at[t],
                device_id=(my,),
                device_id_type=pl.DeviceIdType.MESH,
            ).wait_recv()
            out_ref[pl.ds(src * M_PER, M_PER), :] = (
                recv8[src].astype(jnp.float32) * scale
            ).astype(jnp.bfloat16)

        for rdma in bf_sends + amax_sends + a2a_sends:
            rdma.wait_send()

    return pl.pallas_call(
        body,
        out_shape=jax.ShapeDtypeStruct((N_DEV * M_PER, N_PER), jnp.bfloat16),
        in_specs=[
            pl.BlockSpec(memory_space=pltpu.VMEM),
            pl.BlockSpec(memory_space=pltpu.MemorySpace.HBM),
        ],
        out_specs=pl.BlockSpec(memory_space=pltpu.VMEM),
        scratch_shapes=[
            pltpu.VMEM((2, K, N_PER), jnp.float32),
            pltpu.VMEM((N_BF, M_PER, N_PER), jnp.bfloat16),
            pltpu.VMEM((N_DEV, M_PER, N_PER), jnp.float32),
            pltpu.VMEM((N_DEV, M_PER, N_PER), jnp.float8_e4m3fn),
            pltpu.VMEM((N_DEV, M_PER, N_PER), jnp.float8_e4m3fn),
            pltpu.VMEM((N_BF, M_PER, N_PER), jnp.bfloat16),
            pltpu.VMEM((N_DEV, 8, 128), jnp.float32),
            pltpu.SemaphoreType.DMA((2,)),
            pltpu.SemaphoreType.DMA((N_DEV,)),
            pltpu.SemaphoreType.DMA((N_DEV,)),
            pltpu.SemaphoreType.DMA((N_BF,)),
            pltpu.SemaphoreType.DMA((N_BF,)),
            pltpu.SemaphoreType.DMA((N_DEV,)),
            pltpu.SemaphoreType.DMA((N_DEV,)),
        ],
        compiler_params=pltpu.CompilerParams(
            collective_id=0, vmem_limit_bytes=48 * 1024 * 1024,
        ),
    )(x, w_mat)
